# baseline (speedup 1.0000x reference)
import functools

import jax
import jax.numpy as jnp
from jax import lax
from jax.experimental import pallas as pl
from jax.experimental.pallas import tpu as pltpu
from jax.experimental.pallas import tpu_sc as plsc

D_EMBED = 64
NTOKENS = 1000000
BATCH = 4096
HIST = 50
TOTAL = BATCH * HIST
SCALE = 8.0

NUM_WORKERS = 32
PER_WORKER = TOTAL // NUM_WORKERS

SUB = 128
NSUB = 5
CHUNK = SUB * NSUB
NCHUNKS = PER_WORKER // CHUNK

_TWO23 = 8388608.0


def _body(x_hbm, lut_hbm, out_hbm, xv, idxv, rows, sem):
    wid = lax.axis_index("s") * 2 + lax.axis_index("c")
    base = wid * PER_WORKER

    def chunk_body(i, carry):
        cbase = base + i * CHUNK
        pltpu.sync_copy(x_hbm.at[pl.ds(cbase, CHUNK)], xv)
        for j in range(CHUNK // 16):
            v = xv[pl.ds(j * 16, 16)]
            t = v * jnp.float32(NTOKENS - 1)
            r = (t + _TWO23) - _TWO23
            s_idx = j // (SUB // 16)
            lane = (j % (SUB // 16)) * 16
            idxv[s_idx, pl.ds(lane, 16)] = r.astype(jnp.int32)
        copies = []
        for s in range(NSUB):
            copies.append(
                pltpu.async_copy(lut_hbm.at[idxv.at[s]], rows.at[s], sem)
            )
        for c in copies:
            c.wait()
        def scale_body(r4, carry2):
            for u in range(4):
                row = r4 * 4 + u
                s_idx = row // SUB
                r_idx = row % SUB
                for k in range(D_EMBED // 16):
                    sl = rows[s_idx, r_idx, pl.ds(k * 16, 16)]
                    rows[s_idx, r_idx, pl.ds(k * 16, 16)] = sl * SCALE
            return carry2

        lax.fori_loop(0, CHUNK // 4, scale_body, 0, unroll=False)
        for s in range(NSUB):
            pltpu.sync_copy(rows.at[s], out_hbm.at[pl.ds(cbase + s * SUB, SUB)])
        return carry

    lax.fori_loop(0, NCHUNKS, chunk_body, 0, unroll=False)


_mesh = plsc.VectorSubcoreMesh(core_axis_name="c", subcore_axis_name="s")

_emb = functools.partial(
    pl.kernel,
    out_type=jax.ShapeDtypeStruct((TOTAL, D_EMBED), jnp.float32),
    mesh=_mesh,
    scratch_types=[
        pltpu.VMEM((CHUNK,), jnp.float32),
        pltpu.VMEM((NSUB, SUB), jnp.int32),
        pltpu.VMEM((NSUB, SUB, D_EMBED), jnp.float32),
        pltpu.SemaphoreType.DMA,
    ],
    compiler_params=pltpu.CompilerParams(use_tc_tiling_on_sc=False),
)(_body)


def kernel(x, lut):
    xf = x.reshape(TOTAL)
    out = _emb(xf, lut)
    return out.reshape(BATCH, HIST, D_EMBED)
